# baseline (device time: 116898 ns/iter reference)
import jax
import jax.numpy as jnp
from jax import lax
from jax.experimental import pallas as pl
from jax.experimental.pallas import tpu as pltpu

N_DEV = 16


def kernel(x, w_mat):
    m_per, k = x.shape
    _, n_per = w_mat.shape

    x = x.astype(jnp.bfloat16)
    w = w_mat.astype(jnp.bfloat16)

    def body(x_ref, w_ref, out_ref, comm_ref, send_sems, recv_sems):
        my_pos = lax.axis_index("i")
        left = (my_pos - 1) % N_DEV
        right = (my_pos + 1) % N_DEV

        barrier_sem = pltpu.get_barrier_semaphore()
        for nbr in (left, right):
            pl.semaphore_signal(
                barrier_sem, inc=1,
                device_id=(nbr,), device_id_type=pl.DeviceIdType.MESH,
            )
        pl.semaphore_wait(barrier_sem, 2)

        hops = []
        for h in range(N_DEV - 1):
            src = x_ref if h == 0 else comm_ref.at[h - 1]
            hops.append(
                pltpu.make_async_remote_copy(
                    src_ref=src,
                    dst_ref=comm_ref.at[h],
                    send_sem=send_sems.at[h],
                    recv_sem=recv_sems.at[h],
                    device_id=(right,),
                    device_id_type=pl.DeviceIdType.MESH,
                )
            )

        hops[0].start()
        out_ref[pl.ds(my_pos * m_per, m_per), :] = jnp.dot(
            x_ref[...], w_ref[...], preferred_element_type=jnp.float32
        )

        for h in range(N_DEV - 1):
            hops[h].wait_recv()
            if h + 1 < N_DEV - 1:
                hops[h + 1].start()
            origin = (my_pos - h - 1) % N_DEV
            out_ref[pl.ds(origin * m_per, m_per), :] = jnp.dot(
                comm_ref[h], w_ref[...], preferred_element_type=jnp.float32
            )

        for h in range(N_DEV - 1):
            hops[h].wait_send()

    return pl.pallas_call(
        body,
        out_shape=jax.ShapeDtypeStruct((N_DEV * m_per, n_per), jnp.float32),
        in_specs=[
            pl.BlockSpec(memory_space=pltpu.VMEM),
            pl.BlockSpec(memory_space=pltpu.VMEM),
        ],
        out_specs=pl.BlockSpec(memory_space=pltpu.VMEM),
        scratch_shapes=[
            pltpu.VMEM((N_DEV - 1, m_per, k), jnp.bfloat16),
            pltpu.SemaphoreType.DMA((N_DEV - 1,)),
            pltpu.SemaphoreType.DMA((N_DEV - 1,)),
        ],
        compiler_params=pltpu.CompilerParams(collective_id=0),
    )(x, w)


# device time: 64992 ns/iter; 1.7987x vs baseline; 1.7987x over previous
import jax
import jax.numpy as jnp
from jax import lax
from jax.experimental import pallas as pl
from jax.experimental.pallas import tpu as pltpu

N_DEV = 16
N_CW = 8
N_CCW = 7

HAM = [0, 4, 8, 12, 13, 9, 5, 1, 2, 6, 10, 14, 15, 11, 7, 3]
INV = [HAM.index(p) for p in range(N_DEV)]


def _sel(table, idx):
    acc = jnp.int32(table[0])
    for k in range(1, N_DEV):
        acc = jnp.where(idx == k, jnp.int32(table[k]), acc)
    return acc


def kernel(x, w_mat):
    m_per, k = x.shape
    _, n_per = w_mat.shape

    x = x.astype(jnp.bfloat16)
    w = w_mat.astype(jnp.bfloat16)

    def body(x_ref, w_ref, out_ref, cw_ref, ccw_ref,
             cw_send, cw_recv, ccw_send, ccw_recv):
        my_pos = lax.axis_index("i")
        r = _sel(INV, my_pos)
        right = _sel([HAM[(i + 1) % N_DEV] for i in range(N_DEV)], r)
        left = _sel([HAM[(i - 1) % N_DEV] for i in range(N_DEV)], r)

        barrier_sem = pltpu.get_barrier_semaphore()
        for nbr in (left, right):
            pl.semaphore_signal(
                barrier_sem, inc=1,
                device_id=(nbr,), device_id_type=pl.DeviceIdType.MESH,
            )
        pl.semaphore_wait(barrier_sem, 2)

        cw = []
        for h in range(N_CW):
            cw.append(pltpu.make_async_remote_copy(
                src_ref=x_ref if h == 0 else cw_ref.at[h - 1],
                dst_ref=cw_ref.at[h],
                send_sem=cw_send.at[h],
                recv_sem=cw_recv.at[h],
                device_id=(right,),
                device_id_type=pl.DeviceIdType.MESH,
            ))
        ccw = []
        for h in range(N_CCW):
            ccw.append(pltpu.make_async_remote_copy(
                src_ref=x_ref if h == 0 else ccw_ref.at[h - 1],
                dst_ref=ccw_ref.at[h],
                send_sem=ccw_send.at[h],
                recv_sem=ccw_recv.at[h],
                device_id=(left,),
                device_id_type=pl.DeviceIdType.MESH,
            ))

        cw[0].start()
        ccw[0].start()
        out_ref[pl.ds(my_pos * m_per, m_per), :] = jnp.dot(
            x_ref[...], w_ref[...], preferred_element_type=jnp.float32
        )

        for h in range(N_CW):
            cw[h].wait_recv()
            if h + 1 < N_CW:
                cw[h + 1].start()
            if h < N_CCW:
                ccw[h].wait_recv()
                if h + 1 < N_CCW:
                    ccw[h + 1].start()

            origin_cw = _sel(HAM, (r - h - 1) % N_DEV)
            out_ref[pl.ds(origin_cw * m_per, m_per), :] = jnp.dot(
                cw_ref[h], w_ref[...], preferred_element_type=jnp.float32
            )
            if h < N_CCW:
                origin_ccw = _sel(HAM, (r + h + 1) % N_DEV)
                out_ref[pl.ds(origin_ccw * m_per, m_per), :] = jnp.dot(
                    ccw_ref[h], w_ref[...], preferred_element_type=jnp.float32
                )

        for h in range(N_CW):
            cw[h].wait_send()
        for h in range(N_CCW):
            ccw[h].wait_send()

    return pl.pallas_call(
        body,
        out_shape=jax.ShapeDtypeStruct((N_DEV * m_per, n_per), jnp.float32),
        in_specs=[
            pl.BlockSpec(memory_space=pltpu.VMEM),
            pl.BlockSpec(memory_space=pltpu.VMEM),
        ],
        out_specs=pl.BlockSpec(memory_space=pltpu.VMEM),
        scratch_shapes=[
            pltpu.VMEM((N_CW, m_per, k), jnp.bfloat16),
            pltpu.VMEM((N_CCW, m_per, k), jnp.bfloat16),
            pltpu.SemaphoreType.DMA((N_CW,)),
            pltpu.SemaphoreType.DMA((N_CW,)),
            pltpu.SemaphoreType.DMA((N_CCW,)),
            pltpu.SemaphoreType.DMA((N_CCW,)),
        ],
        compiler_params=pltpu.CompilerParams(collective_id=0),
    )(x, w)


# device time: 53195 ns/iter; 2.1975x vs baseline; 1.2218x over previous
import jax
import jax.numpy as jnp
from jax import lax
from jax.experimental import pallas as pl
from jax.experimental.pallas import tpu as pltpu

N_DEV = 16
N_CW = 8
N_CCW = 7

HAM = [0, 4, 8, 12, 13, 9, 5, 1, 2, 6, 10, 14, 15, 11, 7, 3]
INV = [HAM.index(p) for p in range(N_DEV)]


def _sel(table, idx):
    acc = jnp.int32(table[0])
    for k in range(1, N_DEV):
        acc = jnp.where(idx == k, jnp.int32(table[k]), acc)
    return acc


def kernel(x, w_mat):
    m_per, k = x.shape
    _, n_per = w_mat.shape
    m_half = m_per // 2

    x = x.astype(jnp.bfloat16)
    w = w_mat.astype(jnp.bfloat16)

    def body(x_ref, w_ref, out_ref, cw_ref, ccw_ref,
             cw_send, cw_recv, ccw_send, ccw_recv):
        my_pos = lax.axis_index("i")
        r = _sel(INV, my_pos)
        right = _sel([HAM[(i + 1) % N_DEV] for i in range(N_DEV)], r)
        left = _sel([HAM[(i - 1) % N_DEV] for i in range(N_DEV)], r)

        barrier_sem = pltpu.get_barrier_semaphore()
        for nbr in (left, right):
            pl.semaphore_signal(
                barrier_sem, inc=1,
                device_id=(nbr,), device_id_type=pl.DeviceIdType.MESH,
            )
        pl.semaphore_wait(barrier_sem, 2)

        def make(stream_ref, send_sems, recv_sems, n_hops, dev):
            descs = []
            for h in range(n_hops):
                per_half = []
                for s in range(2):
                    rows = pl.ds(s * m_half, m_half)
                    src = x_ref.at[rows, :] if h == 0 else stream_ref.at[h - 1, rows, :]
                    per_half.append(pltpu.make_async_remote_copy(
                        src_ref=src,
                        dst_ref=stream_ref.at[h, rows, :],
                        send_sem=send_sems.at[h, s],
                        recv_sem=recv_sems.at[h, s],
                        device_id=(dev,),
                        device_id_type=pl.DeviceIdType.MESH,
                    ))
                descs.append(per_half)
            return descs

        cw = make(cw_ref, cw_send, cw_recv, N_CW, right)
        ccw = make(ccw_ref, ccw_send, ccw_recv, N_CCW, left)

        for s in range(2):
            cw[0][s].start()
            ccw[0][s].start()
        out_ref[pl.ds(my_pos * m_per, m_per), :] = jnp.dot(
            x_ref[...], w_ref[...], preferred_element_type=jnp.float32
        )

        for h in range(N_CW):
            cw[h][0].wait_recv()
            if h + 1 < N_CW:
                cw[h + 1][0].start()
            if h < N_CCW:
                ccw[h][0].wait_recv()
                if h + 1 < N_CCW:
                    ccw[h + 1][0].start()
            cw[h][1].wait_recv()
            if h + 1 < N_CW:
                cw[h + 1][1].start()
            if h < N_CCW:
                ccw[h][1].wait_recv()
                if h + 1 < N_CCW:
                    ccw[h + 1][1].start()

            origin_cw = _sel(HAM, (r - h - 1) % N_DEV)
            out_ref[pl.ds(origin_cw * m_per, m_per), :] = jnp.dot(
                cw_ref[h], w_ref[...], preferred_element_type=jnp.float32
            )
            if h < N_CCW:
                origin_ccw = _sel(HAM, (r + h + 1) % N_DEV)
                out_ref[pl.ds(origin_ccw * m_per, m_per), :] = jnp.dot(
                    ccw_ref[h], w_ref[...], preferred_element_type=jnp.float32
                )

        for h in range(N_CW):
            for s in range(2):
                cw[h][s].wait_send()
        for h in range(N_CCW):
            for s in range(2):
                ccw[h][s].wait_send()

    return pl.pallas_call(
        body,
        out_shape=jax.ShapeDtypeStruct((N_DEV * m_per, n_per), jnp.float32),
        in_specs=[
            pl.BlockSpec(memory_space=pltpu.VMEM),
            pl.BlockSpec(memory_space=pltpu.VMEM),
        ],
        out_specs=pl.BlockSpec(memory_space=pltpu.VMEM),
        scratch_shapes=[
            pltpu.VMEM((N_CW, m_per, k), jnp.bfloat16),
            pltpu.VMEM((N_CCW, m_per, k), jnp.bfloat16),
            pltpu.SemaphoreType.DMA((N_CW, 2)),
            pltpu.SemaphoreType.DMA((N_CW, 2)),
            pltpu.SemaphoreType.DMA((N_CCW, 2)),
            pltpu.SemaphoreType.DMA((N_CCW, 2)),
        ],
        compiler_params=pltpu.CompilerParams(collective_id=0),
    )(x, w)


# device time: 51842 ns/iter; 2.2549x vs baseline; 1.0261x over previous
import jax
import jax.numpy as jnp
from jax import lax
from jax.experimental import pallas as pl
from jax.experimental.pallas import tpu as pltpu

N_DEV = 16
N_HOP = 8
N_Q = 4

Q_CW = [list(range(N_Q))] * 7 + [[0, 1]]
Q_CCW = [list(range(N_Q))] * 7 + [[2, 3]]

HAM = [0, 4, 8, 12, 13, 9, 5, 1, 2, 6, 10, 14, 15, 11, 7, 3]
INV = [HAM.index(p) for p in range(N_DEV)]


def _sel(table, idx):
    acc = jnp.int32(table[0])
    for k in range(1, N_DEV):
        acc = jnp.where(idx == k, jnp.int32(table[k]), acc)
    return acc


def kernel(x, w_mat):
    m_per, k = x.shape
    _, n_per = w_mat.shape
    m_q = m_per // N_Q

    x = x.astype(jnp.bfloat16)
    w = w_mat.astype(jnp.bfloat16)

    def body(x_ref, w_ref, out_ref, cw_ref, ccw_ref,
             cw_send, cw_recv, ccw_send, ccw_recv):
        my_pos = lax.axis_index("i")
        r = _sel(INV, my_pos)
        right = _sel([HAM[(i + 1) % N_DEV] for i in range(N_DEV)], r)
        left = _sel([HAM[(i - 1) % N_DEV] for i in range(N_DEV)], r)

        barrier_sem = pltpu.get_barrier_semaphore()
        for nbr in (left, right):
            pl.semaphore_signal(
                barrier_sem, inc=1,
                device_id=(nbr,), device_id_type=pl.DeviceIdType.MESH,
            )
        pl.semaphore_wait(barrier_sem, 2)

        def make(stream_ref, send_sems, recv_sems, hop_qs, dev):
            descs = []
            for h, qs in enumerate(hop_qs):
                per_q = {}
                for q in qs:
                    rows = pl.ds(q * m_q, m_q)
                    src = (x_ref.at[rows, :] if h == 0
                           else stream_ref.at[h - 1, rows, :])
                    per_q[q] = pltpu.make_async_remote_copy(
                        src_ref=src,
                        dst_ref=stream_ref.at[h, rows, :],
                        send_sem=send_sems.at[h, q],
                        recv_sem=recv_sems.at[h, q],
                        device_id=(dev,),
                        device_id_type=pl.DeviceIdType.MESH,
                    )
                descs.append(per_q)
            return descs

        cw = make(cw_ref, cw_send, cw_recv, Q_CW, right)
        ccw = make(ccw_ref, ccw_send, ccw_recv, Q_CCW, left)

        for q in range(N_Q):
            cw[0][q].start()
            ccw[0][q].start()
        out_ref[pl.ds(my_pos * m_per, m_per), :] = jnp.dot(
            x_ref[...], w_ref[...], preferred_element_type=jnp.float32
        )

        for h in range(N_HOP):
            for q in range(N_Q):
                if q in Q_CW[h]:
                    cw[h][q].wait_recv()
                    if h + 1 < N_HOP and q in Q_CW[h + 1]:
                        cw[h + 1][q].start()
                if q in Q_CCW[h]:
                    ccw[h][q].wait_recv()
                    if h + 1 < N_HOP and q in Q_CCW[h + 1]:
                        ccw[h + 1][q].start()

            if h < N_HOP - 1:
                origin_cw = _sel(HAM, (r - h - 1) % N_DEV)
                out_ref[pl.ds(origin_cw * m_per, m_per), :] = jnp.dot(
                    cw_ref[h], w_ref[...], preferred_element_type=jnp.float32
                )
                origin_ccw = _sel(HAM, (r + h + 1) % N_DEV)
                out_ref[pl.ds(origin_ccw * m_per, m_per), :] = jnp.dot(
                    ccw_ref[h], w_ref[...], preferred_element_type=jnp.float32
                )
            else:
                anti = _sel(HAM, (r + N_HOP) % N_DEV)
                half = N_Q // 2 * m_q
                out_ref[pl.ds(anti * m_per, half), :] = jnp.dot(
                    cw_ref[h, :half, :], w_ref[...],
                    preferred_element_type=jnp.float32,
                )
                out_ref[pl.ds(anti * m_per + half, half), :] = jnp.dot(
                    ccw_ref[h, half:, :], w_ref[...],
                    preferred_element_type=jnp.float32,
                )

        for h in range(N_HOP):
            for q in Q_CW[h]:
                cw[h][q].wait_send()
            for q in Q_CCW[h]:
                ccw[h][q].wait_send()

    return pl.pallas_call(
        body,
        out_shape=jax.ShapeDtypeStruct((N_DEV * m_per, n_per), jnp.float32),
        in_specs=[
            pl.BlockSpec(memory_space=pltpu.VMEM),
            pl.BlockSpec(memory_space=pltpu.VMEM),
        ],
        out_specs=pl.BlockSpec(memory_space=pltpu.VMEM),
        scratch_shapes=[
            pltpu.VMEM((N_HOP, m_per, k), jnp.bfloat16),
            pltpu.VMEM((N_HOP, m_per, k), jnp.bfloat16),
            pltpu.SemaphoreType.DMA((N_HOP, N_Q)),
            pltpu.SemaphoreType.DMA((N_HOP, N_Q)),
            pltpu.SemaphoreType.DMA((N_HOP, N_Q)),
            pltpu.SemaphoreType.DMA((N_HOP, N_Q)),
        ],
        compiler_params=pltpu.CompilerParams(collective_id=0),
    )(x, w)
